# Initial kernel scaffold; baseline (speedup 1.0000x reference)
import jax
import jax.numpy as jnp
from jax.experimental import pallas as pl


def kernel(src, pos, src_shape, src_start_idx, ref_windows, params):
    raise NotImplementedError("write your pallas kernel here")


# speedup vs baseline: 1.1598x; 1.1598x over previous
import functools
import math

import jax
import jax.numpy as jnp
from jax.experimental import pallas as pl
from jax.experimental.pallas import tpu as pltpu

_B, _L, _D = 2, 16384, 256
_R = 4
_NQ = 1000
_NUM_LAYERS = 3
_D_FF = 256
_NUM_CLASSES = 3
_TILE = 2048
_N = _B * _L


def _inverse_sigmoid(x, eps=1e-5):
    x = jnp.clip(x, 0.0, 1.0)
    return jnp.log(jnp.clip(x, eps, None) / jnp.clip(1.0 - x, eps, None))


def _proposal_pos_embed(proposals, d_model):
    num_pos_feats = d_model // 2
    dim_t = jnp.arange(num_pos_feats, dtype=jnp.float32)
    dim_t = 10000.0 ** (2.0 * jnp.floor(dim_t / 2.0) / num_pos_feats)
    p = proposals * (2.0 * math.pi)
    pos = p[..., None] / dim_t
    pos = jnp.stack([jnp.sin(pos[..., 0::2]), jnp.cos(pos[..., 1::2])], axis=-1)
    return pos.reshape(pos.shape[0], pos.shape[1], -1)


_CSPEC = pl.BlockSpec((_D, _TILE), lambda i: (0, i))
_RWSPEC = pl.BlockSpec((_R, _TILE), lambda i: (0, i))


def _full_spec(a):
    return pl.BlockSpec(a.shape, lambda i: tuple(0 for _ in a.shape))


def _t_layer_body(x_ref, p_ref, w1_ref, b1_ref, w2_ref, b2_ref, out_ref):
    x = x_ref[...]
    h = x + p_ref[...]
    h = jnp.maximum(
        jnp.dot(w1_ref[...], h, preferred_element_type=jnp.float32)
        + b1_ref[...], 0.0)
    h = jnp.dot(w2_ref[...], h, preferred_element_type=jnp.float32) + b2_ref[...]
    out_ref[...] = x + h


def _t_final_body(x_ref, rwx_ref, rwy_ref, wc_ref, bc_ref, logit_ref):
    lg = jnp.dot(wc_ref[...], x_ref[...],
                 preferred_element_type=jnp.float32) + bc_ref[...]
    valid = ((rwx_ref[...] > 0.001) & (rwx_ref[...] < 0.999)
             & (rwy_ref[...] > 0.001) & (rwy_ref[...] < 0.999))
    logit_ref[...] = jnp.where(valid, lg, -65504.0)


def _t_ln(xt, g, b, eps=1e-5):
    mu = jnp.mean(xt, axis=0, keepdims=True)
    var = jnp.mean((xt - mu) ** 2, axis=0, keepdims=True)
    return (xt - mu) / jnp.sqrt(var + eps) * g[:, None] + b[:, None]


def _run_encoder(src, pos, ref_windows, params):
    lay = params['layers']
    grid = (_N // _TILE,)
    pt = jnp.transpose(pos.reshape(_N, _D))
    xt = jnp.transpose(src.reshape(_N, _D))

    for i in range(_NUM_LAYERS):
        lp = lay[i]
        w1t, b1c = jnp.transpose(lp['W1']), lp['b1'][:, None]
        w2t, b2c = jnp.transpose(lp['W2']), lp['b2'][:, None]
        xpre_t = pl.pallas_call(
            _t_layer_body,
            grid=grid,
            in_specs=[_CSPEC, _CSPEC, _full_spec(w1t), _full_spec(b1c),
                      _full_spec(w2t), _full_spec(b2c)],
            out_specs=_CSPEC,
            out_shape=jax.ShapeDtypeStruct((_D, _N), jnp.float32),
        )(xt, pt, w1t, b1c, w2t, b2c)
        xt = _t_ln(xpre_t, lp['g'], lp['b'])

    wct = jnp.transpose(params['cls']['W'][:, 0::_NUM_CLASSES])
    bcc = params['cls']['b'][0::_NUM_CLASSES][:, None]
    rwxt = jnp.transpose(ref_windows[..., 0].reshape(_N, _R))
    rwyt = jnp.transpose(ref_windows[..., 1].reshape(_N, _R))
    logit_t = pl.pallas_call(
        _t_final_body,
        grid=grid,
        in_specs=[_CSPEC, _RWSPEC, _RWSPEC, _full_spec(wct), _full_spec(bcc)],
        out_specs=_RWSPEC,
        out_shape=jax.ShapeDtypeStruct((_R, _N), jnp.float32),
    )(xt, rwxt, rwyt, wct, bcc)

    output = jnp.transpose(xt).reshape(_B, _L, _D)
    logits = jnp.transpose(logit_t).reshape(_B, _L * _R)
    return output, logits


def kernel(src, pos, src_shape, src_start_idx, ref_windows, params):
    output, logits = _run_encoder(src, pos, ref_windows, params)

    _, indexes = jax.lax.top_k(logits, _NQ)

    idx_e = indexes // _R
    rows = jnp.take_along_axis(
        output, jnp.broadcast_to(idx_e[:, :, None], (_B, _NQ, _D)), axis=1)

    bb = params['bbox']
    h = jnp.maximum(rows @ bb['W1'] + bb['b1'], 0.0)
    h = jnp.maximum(h @ bb['W2'] + bb['b2'], 0.0)
    tmp = (h @ bb['W3'] + bb['b3']).reshape(_B, _NQ, _R, 7)
    r_sel = indexes % _R
    tmp_sel = jnp.take_along_axis(
        tmp, jnp.broadcast_to(r_sel[:, :, None, None], (_B, _NQ, 1, 7)),
        axis=2)[:, :, 0]

    rw_flat = ref_windows.reshape(_B, _L * _R, 5)
    rw_g = jnp.take_along_axis(
        rw_flat, jnp.broadcast_to(indexes[:, :, None], (_B, _NQ, 5)), axis=1)

    tmp_box = tmp_sel[..., :5] + _inverse_sigmoid(rw_g)
    out_ref_windows = jax.nn.sigmoid(
        jnp.concatenate([tmp_box, tmp_sel[..., 5:]], axis=-1))

    pe = _proposal_pos_embed(out_ref_windows[..., :2], _D)
    se = _proposal_pos_embed(out_ref_windows[..., 2:4], _D)
    re = _proposal_pos_embed(
        jnp.stack([out_ref_windows[..., 4], out_ref_windows[..., 4]], axis=-1),
        _D)
    out_pos = pe + se + re

    ep = params['enc']
    oe = rows @ ep['W'] + ep['b']
    mu = jnp.mean(oe, axis=-1, keepdims=True)
    var = jnp.mean((oe - mu) ** 2, axis=-1, keepdims=True)
    out_embed = (oe - mu) / jnp.sqrt(var + 1e-5) * ep['g'] + ep['beta']

    return (output, out_embed, out_ref_windows, out_pos)


# speedup vs baseline: 1.1600x; 1.0002x over previous
import functools
import math

import jax
import jax.numpy as jnp
from jax.experimental import pallas as pl
from jax.experimental.pallas import tpu as pltpu

_B, _L, _D = 2, 16384, 256
_R = 4
_NQ = 1000
_NUM_LAYERS = 3
_D_FF = 256
_NUM_CLASSES = 3
_TILE = 2048
_N = _B * _L


def _inverse_sigmoid(x, eps=1e-5):
    x = jnp.clip(x, 0.0, 1.0)
    return jnp.log(jnp.clip(x, eps, None) / jnp.clip(1.0 - x, eps, None))


def _proposal_pos_embed(proposals, d_model):
    num_pos_feats = d_model // 2
    dim_t = jnp.arange(num_pos_feats, dtype=jnp.float32)
    dim_t = 10000.0 ** (2.0 * jnp.floor(dim_t / 2.0) / num_pos_feats)
    p = proposals * (2.0 * math.pi)
    pos = p[..., None] / dim_t
    pos = jnp.stack([jnp.sin(pos[..., 0::2]), jnp.cos(pos[..., 1::2])], axis=-1)
    return pos.reshape(pos.shape[0], pos.shape[1], -1)


_CSPEC = pl.BlockSpec((_D, _TILE), lambda i: (0, i))
_RWSPEC = pl.BlockSpec((_R, _TILE), lambda i: (0, i))


def _full_spec(a):
    return pl.BlockSpec(a.shape, lambda i: tuple(0 for _ in a.shape))


def _t_layer_body(x_ref, p_ref, w1_ref, b1_ref, w2_ref, b2_ref, out_ref):
    x = x_ref[...]
    h = x + p_ref[...]
    h = jnp.maximum(
        jnp.dot(w1_ref[...], h, preferred_element_type=jnp.float32)
        + b1_ref[...], 0.0)
    h = jnp.dot(w2_ref[...], h, preferred_element_type=jnp.float32) + b2_ref[...]
    out_ref[...] = x + h


def _t_final_body(x_ref, rwx_ref, rwy_ref, wc_ref, bc_ref, logit_ref):
    lg = jnp.dot(wc_ref[...], x_ref[...],
                 preferred_element_type=jnp.float32) + bc_ref[...]
    valid = ((rwx_ref[...] > 0.001) & (rwx_ref[...] < 0.999)
             & (rwy_ref[...] > 0.001) & (rwy_ref[...] < 0.999))
    logit_ref[...] = jnp.where(valid, lg, -65504.0)


def _t_ln(xt, g, b, eps=1e-5):
    mu = jnp.mean(xt, axis=0, keepdims=True)
    var = jnp.mean((xt - mu) ** 2, axis=0, keepdims=True)
    return (xt - mu) / jnp.sqrt(var + eps) * g[:, None] + b[:, None]


def _run_encoder(src, pos, ref_windows, params):
    lay = params['layers']
    grid = (_N // _TILE,)
    pt = jnp.transpose(pos.reshape(_N, _D))
    xt = jnp.transpose(src.reshape(_N, _D))

    for i in range(_NUM_LAYERS):
        lp = lay[i]
        w1t, b1c = jnp.transpose(lp['W1']), lp['b1'][:, None]
        w2t, b2c = jnp.transpose(lp['W2']), lp['b2'][:, None]
        xpre_t = pl.pallas_call(
            _t_layer_body,
            grid=grid,
            in_specs=[_CSPEC, _CSPEC, _full_spec(w1t), _full_spec(b1c),
                      _full_spec(w2t), _full_spec(b2c)],
            out_specs=_CSPEC,
            out_shape=jax.ShapeDtypeStruct((_D, _N), jnp.float32),
        )(xt, pt, w1t, b1c, w2t, b2c)
        xt = _t_ln(xpre_t, lp['g'], lp['b'])

    wct = jnp.transpose(params['cls']['W'][:, 0::_NUM_CLASSES])
    bcc = params['cls']['b'][0::_NUM_CLASSES][:, None]
    rwxt = jnp.transpose(ref_windows[..., 0].reshape(_N, _R))
    rwyt = jnp.transpose(ref_windows[..., 1].reshape(_N, _R))
    logit_t = pl.pallas_call(
        _t_final_body,
        grid=grid,
        in_specs=[_CSPEC, _RWSPEC, _RWSPEC, _full_spec(wct), _full_spec(bcc)],
        out_specs=_RWSPEC,
        out_shape=jax.ShapeDtypeStruct((_R, _N), jnp.float32),
    )(xt, rwxt, rwyt, wct, bcc)

    output = jnp.transpose(xt).reshape(_B, _L, _D)
    logits = jnp.transpose(logit_t).reshape(_B, _L * _R)
    return output, logits


def kernel(src, pos, src_shape, src_start_idx, ref_windows, params):
    output, logits = _run_encoder(src, pos, ref_windows, params)

    _, indexes = jax.lax.top_k(logits, _NQ)

    idx_e = indexes // _R
    rows = jnp.take_along_axis(
        output, jnp.broadcast_to(idx_e[:, :, None], (_B, _NQ, _D)), axis=1)

    bb = params['bbox']
    rows2 = rows.reshape(_B * _NQ, _D)
    h = jnp.maximum(rows2 @ bb['W1'] + bb['b1'], 0.0)
    h = jnp.maximum(h @ bb['W2'] + bb['b2'], 0.0)
    tmp = (h @ bb['W3'] + bb['b3']).reshape(_B, _NQ, _R, 7)
    r_sel = indexes % _R
    tmp_sel = jnp.take_along_axis(
        tmp, jnp.broadcast_to(r_sel[:, :, None, None], (_B, _NQ, 1, 7)),
        axis=2)[:, :, 0]

    rw_flat = ref_windows.reshape(_B, _L * _R, 5)
    rw_g = jnp.take_along_axis(
        rw_flat, jnp.broadcast_to(indexes[:, :, None], (_B, _NQ, 5)), axis=1)

    tmp_box = tmp_sel[..., :5] + _inverse_sigmoid(rw_g)
    out_ref_windows = jax.nn.sigmoid(
        jnp.concatenate([tmp_box, tmp_sel[..., 5:]], axis=-1))

    pe = _proposal_pos_embed(out_ref_windows[..., :2], _D)
    se = _proposal_pos_embed(out_ref_windows[..., 2:4], _D)
    re = _proposal_pos_embed(
        jnp.stack([out_ref_windows[..., 4], out_ref_windows[..., 4]], axis=-1),
        _D)
    out_pos = pe + se + re

    ep = params['enc']
    oe = (rows.reshape(_B * _NQ, _D) @ ep['W']).reshape(_B, _NQ, _D) + ep['b']
    mu = jnp.mean(oe, axis=-1, keepdims=True)
    var = jnp.mean((oe - mu) ** 2, axis=-1, keepdims=True)
    out_embed = (oe - mu) / jnp.sqrt(var + 1e-5) * ep['g'] + ep['beta']

    return (output, out_embed, out_ref_windows, out_pos)


# speedup vs baseline: 8.8539x; 7.6327x over previous
import functools
import math

import jax
import jax.numpy as jnp
from jax.experimental import pallas as pl
from jax.experimental.pallas import tpu as pltpu

_B, _L, _D = 2, 16384, 256
_R = 4
_NQ = 1000
_NUM_LAYERS = 3
_D_FF = 256
_NUM_CLASSES = 3
_TILE = 2048
_N = _B * _L


def _inverse_sigmoid(x, eps=1e-5):
    x = jnp.clip(x, 0.0, 1.0)
    return jnp.log(jnp.clip(x, eps, None) / jnp.clip(1.0 - x, eps, None))


def _proposal_pos_embed(proposals, d_model):
    num_pos_feats = d_model // 2
    dim_t = jnp.arange(num_pos_feats, dtype=jnp.float32)
    dim_t = 10000.0 ** (2.0 * jnp.floor(dim_t / 2.0) / num_pos_feats)
    p = proposals * (2.0 * math.pi)
    pos = p[..., None] / dim_t
    pos = jnp.stack([jnp.sin(pos[..., 0::2]), jnp.cos(pos[..., 1::2])], axis=-1)
    return pos.reshape(pos.shape[0], pos.shape[1], -1)


_CSPEC = pl.BlockSpec((_D, _TILE), lambda i: (0, i))
_RWSPEC = pl.BlockSpec((_R, _TILE), lambda i: (0, i))


def _full_spec(a):
    return pl.BlockSpec(a.shape, lambda i: tuple(0 for _ in a.shape))


def _t_layer_body(x_ref, p_ref, w1_ref, b1_ref, w2_ref, b2_ref, out_ref):
    x = x_ref[...]
    h = x + p_ref[...]
    h = jnp.maximum(
        jnp.dot(w1_ref[...], h, preferred_element_type=jnp.float32)
        + b1_ref[...], 0.0)
    h = jnp.dot(w2_ref[...], h, preferred_element_type=jnp.float32) + b2_ref[...]
    out_ref[...] = x + h


def _t_final_body(x_ref, rwx_ref, rwy_ref, wc_ref, bc_ref, logit_ref):
    lg = jnp.dot(wc_ref[...], x_ref[...],
                 preferred_element_type=jnp.float32) + bc_ref[...]
    valid = ((rwx_ref[...] > 0.001) & (rwx_ref[...] < 0.999)
             & (rwy_ref[...] > 0.001) & (rwy_ref[...] < 0.999))
    logit_ref[...] = jnp.where(valid, lg, -65504.0)


def _t_ln(xt, g, b, eps=1e-5):
    mu = jnp.mean(xt, axis=0, keepdims=True)
    var = jnp.mean((xt - mu) ** 2, axis=0, keepdims=True)
    return (xt - mu) / jnp.sqrt(var + eps) * g[:, None] + b[:, None]


def _run_encoder(src, pos, ref_windows, params):
    lay = params['layers']
    grid = (_N // _TILE,)
    pt = jnp.transpose(pos.reshape(_N, _D))
    xt = jnp.transpose(src.reshape(_N, _D))

    for i in range(_NUM_LAYERS):
        lp = lay[i]
        w1t, b1c = jnp.transpose(lp['W1']), lp['b1'][:, None]
        w2t, b2c = jnp.transpose(lp['W2']), lp['b2'][:, None]
        xpre_t = pl.pallas_call(
            _t_layer_body,
            grid=grid,
            in_specs=[_CSPEC, _CSPEC, _full_spec(w1t), _full_spec(b1c),
                      _full_spec(w2t), _full_spec(b2c)],
            out_specs=_CSPEC,
            out_shape=jax.ShapeDtypeStruct((_D, _N), jnp.float32),
        )(xt, pt, w1t, b1c, w2t, b2c)
        xt = _t_ln(xpre_t, lp['g'], lp['b'])

    wct = jnp.transpose(params['cls']['W'][:, 0::_NUM_CLASSES])
    bcc = params['cls']['b'][0::_NUM_CLASSES][:, None]
    rwxt = jnp.transpose(ref_windows[..., 0].reshape(_N, _R))
    rwyt = jnp.transpose(ref_windows[..., 1].reshape(_N, _R))
    logit_t = pl.pallas_call(
        _t_final_body,
        grid=grid,
        in_specs=[_CSPEC, _RWSPEC, _RWSPEC, _full_spec(wct), _full_spec(bcc)],
        out_specs=_RWSPEC,
        out_shape=jax.ShapeDtypeStruct((_R, _N), jnp.float32),
    )(xt, rwxt, rwyt, wct, bcc)

    output = jnp.transpose(xt).reshape(_B, _L, _D)
    logits = jnp.transpose(logit_t).reshape(_B, _L * _R)
    return output, logits


def kernel(src, pos, src_shape, src_start_idx, ref_windows, params):
    output, logits = _run_encoder(src, pos, ref_windows, params)

    _, indexes = jax.lax.top_k(logits, _NQ)

    idx_e = indexes // _R
    rows = jnp.take_along_axis(
        output, jnp.broadcast_to(idx_e[:, :, None], (_B, _NQ, _D)), axis=1)

    hi = jax.lax.Precision.HIGHEST
    bb = params['bbox']
    rows2 = rows.reshape(_B * _NQ, _D)
    h = jnp.maximum(jnp.dot(rows2, bb['W1'], precision=hi) + bb['b1'], 0.0)
    h = jnp.maximum(jnp.dot(h, bb['W2'], precision=hi) + bb['b2'], 0.0)
    tmp = (jnp.dot(h, bb['W3'], precision=hi) + bb['b3']).reshape(_B, _NQ, _R, 7)
    r_sel = indexes % _R
    tmp_sel = jnp.take_along_axis(
        tmp, jnp.broadcast_to(r_sel[:, :, None, None], (_B, _NQ, 1, 7)),
        axis=2)[:, :, 0]

    rw_flat = ref_windows.reshape(_B, _L * _R, 5)
    rw_g = jnp.take_along_axis(
        rw_flat, jnp.broadcast_to(indexes[:, :, None], (_B, _NQ, 5)), axis=1)

    tmp_box = tmp_sel[..., :5] + _inverse_sigmoid(rw_g)
    out_ref_windows = jax.nn.sigmoid(
        jnp.concatenate([tmp_box, tmp_sel[..., 5:]], axis=-1))

    pe = _proposal_pos_embed(out_ref_windows[..., :2], _D)
    se = _proposal_pos_embed(out_ref_windows[..., 2:4], _D)
    re = _proposal_pos_embed(
        jnp.stack([out_ref_windows[..., 4], out_ref_windows[..., 4]], axis=-1),
        _D)
    out_pos = pe + se + re

    ep = params['enc']
    oe = jnp.dot(rows.reshape(_B * _NQ, _D), ep['W'],
                 precision=hi).reshape(_B, _NQ, _D) + ep['b']
    mu = jnp.mean(oe, axis=-1, keepdims=True)
    var = jnp.mean((oe - mu) ** 2, axis=-1, keepdims=True)
    out_embed = (oe - mu) / jnp.sqrt(var + 1e-5) * ep['g'] + ep['beta']

    return (output, out_embed, out_ref_windows, out_pos)
